# Initial kernel scaffold; baseline (speedup 1.0000x reference)
import jax
import jax.numpy as jnp
from jax.experimental import pallas as pl


def kernel(image_tensors, feat0, feat1, feat2, feat3, feat4, cls_logits, bbox_regression):
    raise NotImplementedError("write your pallas kernel here")


# speedup vs baseline: 4.8047x; 4.8047x over previous
import functools
import math

import jax
import jax.numpy as jnp
import numpy as np
from jax.experimental import pallas as pl
from jax.experimental.pallas import tpu as pltpu

_IMG_H = 800
_IMG_W = 800
_NUM_CLASSES = 91
_A = 9
_GRIDS = [(100, 100), (50, 50), (25, 25), (13, 13), (7, 7)]
_SIZES = [(32, 40, 50), (64, 80, 101), (128, 161, 203), (256, 322, 406), (512, 645, 812)]
_RATIOS = (0.5, 1.0, 2.0)
_SCORE_THRESH = 0.05
_TOPK = 1000
_NMS_THRESH = 0.5
_DETS = 300
_CLIP = math.log(1000.0 / 16.0)

_N_NMS = 5000
_C = 512
_NPAD = 5120


def _cell_anchors(sizes):
    scales = np.asarray(sizes, dtype=np.float32)
    ratios = np.asarray(_RATIOS, dtype=np.float32)
    h_ratios = np.sqrt(ratios)
    w_ratios = 1.0 / h_ratios
    ws = (w_ratios[:, None] * scales[None, :]).reshape(-1)
    hs = (h_ratios[:, None] * scales[None, :]).reshape(-1)
    base = np.stack([-ws, -hs, ws, hs], axis=1) / 2.0
    return np.round(base).astype(np.float32)


def _make_anchors():
    per_level = []
    for (gh, gw), sz in zip(_GRIDS, _SIZES):
        stride_h = _IMG_H // gh
        stride_w = _IMG_W // gw
        base = _cell_anchors(sz)
        sx = np.arange(gw, dtype=np.float32) * stride_w
        sy = np.arange(gh, dtype=np.float32) * stride_h
        SY, SX = np.meshgrid(sy, sx, indexing="ij")
        shifts = np.stack([SX.reshape(-1), SY.reshape(-1), SX.reshape(-1), SY.reshape(-1)], axis=1)
        a = (shifts[:, None, :] + base[None, :, :]).reshape(-1, 4)
        per_level.append(np.asarray(a, dtype=np.float32))
    return per_level


_ANCHORS_NP = _make_anchors()


def _nms_suppress_kernel(br, bc, out, supp_col):
    nchunks = _NPAD // _C
    ii = jax.lax.broadcasted_iota(jnp.int32, (_C, _C), 0)
    jj = jax.lax.broadcasted_iota(jnp.int32, (_C, _C), 1)
    upper = (jj > ii).astype(jnp.float32)
    ident = (jj == ii).astype(jnp.float32)

    for t in range(nchunks):
        x1t = bc[0, 0:1, t * _C:(t + 1) * _C]
        y1t = bc[0, 1:2, t * _C:(t + 1) * _C]
        x2t = bc[0, 2:3, t * _C:(t + 1) * _C]
        y2t = bc[0, 3:4, t * _C:(t + 1) * _C]
        area_t = (x2t - x1t) * (y2t - y1t)

        cross = jnp.zeros((1, _C), jnp.float32)
        adj = None
        for p in range(t + 1):
            x1p = br[0, p * _C:(p + 1) * _C, 0:1]
            y1p = br[0, p * _C:(p + 1) * _C, 1:2]
            x2p = br[0, p * _C:(p + 1) * _C, 2:3]
            y2p = br[0, p * _C:(p + 1) * _C, 3:4]
            area_p = (x2p - x1p) * (y2p - y1p)
            w = jnp.maximum(jnp.minimum(x2p, x2t) - jnp.maximum(x1p, x1t), 0.0)
            h = jnp.maximum(jnp.minimum(y2p, y2t) - jnp.maximum(y1p, y1t), 0.0)
            inter = w * h
            iou = inter / (area_p + area_t - inter + 1e-7)
            a = (iou > _NMS_THRESH).astype(jnp.float32)
            if p < t:
                kept_p = 1.0 - supp_col[p * _C:(p + 1) * _C, 0:1]
                cross = jnp.maximum(cross, jnp.max(a * kept_p, axis=0, keepdims=True))
            else:
                adj = a * upper

        def to_col(row):
            return jax.lax.dot_general(ident, row,
                                       (((1,), (1,)), ((), ())),
                                       preferred_element_type=jnp.float32)

        def cond(carry):
            return carry[1]

        def body(carry):
            s, _ = carry
            s_col = to_col(s)
            red = jnp.max(adj * (1.0 - s_col), axis=0, keepdims=True)
            new = jnp.maximum(cross, red)
            return new, jnp.any(new != s)

        s, _ = jax.lax.while_loop(cond, body, (cross, jnp.bool_(True)))
        out[0, 0:1, t * _C:(t + 1) * _C] = s
        supp_col[t * _C:(t + 1) * _C, 0:1] = to_col(s)


def _nms_suppress(boxes_off):
    B = boxes_off.shape[0]
    bc = jnp.transpose(boxes_off, (0, 2, 1))
    out = pl.pallas_call(
        _nms_suppress_kernel,
        grid=(B,),
        in_specs=[
            pl.BlockSpec((1, _NPAD, 4), lambda b: (b, 0, 0)),
            pl.BlockSpec((1, 4, _NPAD), lambda b: (b, 0, 0)),
        ],
        out_specs=pl.BlockSpec((1, 1, _NPAD), lambda b: (b, 0, 0)),
        out_shape=jax.ShapeDtypeStruct((B, 1, _NPAD), jnp.float32),
        scratch_shapes=[pltpu.VMEM((_NPAD, 1), jnp.float32)],
    )(boxes_off, bc)
    return out[:, 0, :_N_NMS]


def _decode_clip(rel, anchors):
    w = anchors[..., 2] - anchors[..., 0]
    h = anchors[..., 3] - anchors[..., 1]
    cx = anchors[..., 0] + 0.5 * w
    cy = anchors[..., 1] + 0.5 * h
    dx, dy = rel[..., 0], rel[..., 1]
    dw = jnp.minimum(rel[..., 2], _CLIP)
    dh = jnp.minimum(rel[..., 3], _CLIP)
    pcx = dx * w + cx
    pcy = dy * h + cy
    pw = jnp.exp(dw) * w
    ph = jnp.exp(dh) * h
    x1 = jnp.clip(pcx - 0.5 * pw, 0.0, float(_IMG_W))
    y1 = jnp.clip(pcy - 0.5 * ph, 0.0, float(_IMG_H))
    x2 = jnp.clip(pcx + 0.5 * pw, 0.0, float(_IMG_W))
    y2 = jnp.clip(pcy + 0.5 * ph, 0.0, float(_IMG_H))
    return jnp.stack([x1, y1, x2, y2], axis=-1)


def kernel(image_tensors, feat0, feat1, feat2, feat3, feat4, cls_logits, bbox_regression):
    feats = [feat0, feat1, feat2, feat3, feat4]
    counts = [f.shape[2] * f.shape[3] * _A for f in feats]
    B = cls_logits.shape[0]

    boxes_l, scores_l, labels_l = [], [], []
    off = 0
    for li, cnt in enumerate(counts):
        logits = cls_logits[:, off:off + cnt]
        regress = bbox_regression[:, off:off + cnt]
        off += cnt
        sc = jax.nn.sigmoid(logits).reshape(B, -1)
        masked = jnp.where(sc > _SCORE_THRESH, sc, -1.0)
        top_s, top_i = jax.lax.top_k(masked, _TOPK)
        anchor_idx = top_i // _NUM_CLASSES
        label = top_i % _NUM_CLASSES
        rel = jnp.take_along_axis(regress, anchor_idx[..., None], axis=1)
        anc = jnp.asarray(_ANCHORS_NP[li])[anchor_idx]
        boxes_l.append(_decode_clip(rel, anc))
        scores_l.append(top_s)
        labels_l.append(label)

    boxes = jnp.concatenate(boxes_l, axis=1)
    scores = jnp.concatenate(scores_l, axis=1)
    labels = jnp.concatenate(labels_l, axis=1)

    order = jnp.argsort(-scores, axis=1)
    b = jnp.take_along_axis(boxes, order[..., None], axis=1)
    s = jnp.take_along_axis(scores, order, axis=1)
    l = jnp.take_along_axis(labels, order, axis=1)

    offs = (l.astype(b.dtype) * (float(max(_IMG_H, _IMG_W)) + 1.0))[..., None]
    boff = b + offs
    boff = jnp.pad(boff, ((0, 0), (0, _NPAD - _N_NMS), (0, 0)))
    supp = _nms_suppress(boff)

    kept = jnp.where(supp > 0.5, -1.0, s)
    top_s, top_i = jax.lax.top_k(kept, _DETS)
    out_boxes = jnp.take_along_axis(b, top_i[..., None], axis=1)
    out_labels = jnp.take_along_axis(l, top_i, axis=1)
    det = jnp.concatenate([out_boxes, top_s[..., None]], axis=-1)
    return det, out_labels.astype(jnp.int32)


# speedup vs baseline: 5.8700x; 1.2217x over previous
import functools
import math

import jax
import jax.numpy as jnp
import numpy as np
from jax.experimental import pallas as pl
from jax.experimental.pallas import tpu as pltpu

_IMG_H = 800
_IMG_W = 800
_NUM_CLASSES = 91
_A = 9
_GRIDS = [(100, 100), (50, 50), (25, 25), (13, 13), (7, 7)]
_SIZES = [(32, 40, 50), (64, 80, 101), (128, 161, 203), (256, 322, 406), (512, 645, 812)]
_RATIOS = (0.5, 1.0, 2.0)
_SCORE_THRESH = 0.05
_TOPK = 1000
_NMS_THRESH = 0.5
_DETS = 300
_CLIP = math.log(1000.0 / 16.0)

_N_NMS = 5000
_C = 512
_NPAD = 5120


def _cell_anchors(sizes):
    scales = np.asarray(sizes, dtype=np.float32)
    ratios = np.asarray(_RATIOS, dtype=np.float32)
    h_ratios = np.sqrt(ratios)
    w_ratios = 1.0 / h_ratios
    ws = (w_ratios[:, None] * scales[None, :]).reshape(-1)
    hs = (h_ratios[:, None] * scales[None, :]).reshape(-1)
    base = np.stack([-ws, -hs, ws, hs], axis=1) / 2.0
    return np.round(base).astype(np.float32)


def _make_anchors():
    per_level = []
    for (gh, gw), sz in zip(_GRIDS, _SIZES):
        stride_h = _IMG_H // gh
        stride_w = _IMG_W // gw
        base = _cell_anchors(sz)
        sx = np.arange(gw, dtype=np.float32) * stride_w
        sy = np.arange(gh, dtype=np.float32) * stride_h
        SY, SX = np.meshgrid(sy, sx, indexing="ij")
        shifts = np.stack([SX.reshape(-1), SY.reshape(-1), SX.reshape(-1), SY.reshape(-1)], axis=1)
        a = (shifts[:, None, :] + base[None, :, :]).reshape(-1, 4)
        per_level.append(np.asarray(a, dtype=np.float32))
    return per_level


_ANCHORS_NP = _make_anchors()


def _nms_suppress_kernel(br, bc, out, supp_col):
    nchunks = _NPAD // _C
    ii = jax.lax.broadcasted_iota(jnp.int32, (_C, _C), 0)
    jj = jax.lax.broadcasted_iota(jnp.int32, (_C, _C), 1)
    upper = (jj > ii).astype(jnp.float32)
    ident = (jj == ii).astype(jnp.float32)

    for t in range(nchunks):
        x1t = bc[0, 0:1, t * _C:(t + 1) * _C]
        y1t = bc[0, 1:2, t * _C:(t + 1) * _C]
        x2t = bc[0, 2:3, t * _C:(t + 1) * _C]
        y2t = bc[0, 3:4, t * _C:(t + 1) * _C]
        area_t = (x2t - x1t) * (y2t - y1t)

        cross = jnp.zeros((1, _C), jnp.float32)
        adj = None
        for p in range(t + 1):
            x1p = br[0, p * _C:(p + 1) * _C, 0:1]
            y1p = br[0, p * _C:(p + 1) * _C, 1:2]
            x2p = br[0, p * _C:(p + 1) * _C, 2:3]
            y2p = br[0, p * _C:(p + 1) * _C, 3:4]
            area_p = (x2p - x1p) * (y2p - y1p)
            w = jnp.maximum(jnp.minimum(x2p, x2t) - jnp.maximum(x1p, x1t), 0.0)
            h = jnp.maximum(jnp.minimum(y2p, y2t) - jnp.maximum(y1p, y1t), 0.0)
            inter = w * h
            iou = inter / (area_p + area_t - inter + 1e-7)
            a = (iou > _NMS_THRESH).astype(jnp.float32)
            if p < t:
                kept_p = 1.0 - supp_col[p * _C:(p + 1) * _C, 0:1]
                cross = jnp.maximum(cross, jnp.max(a * kept_p, axis=0, keepdims=True))
            else:
                adj = a * upper

        def to_col(row):
            return jax.lax.dot_general(ident, row,
                                       (((1,), (1,)), ((), ())),
                                       preferred_element_type=jnp.float32)

        def cond(carry):
            return carry[1]

        def body(carry):
            s, _ = carry
            s_col = to_col(s)
            red = jnp.max(adj * (1.0 - s_col), axis=0, keepdims=True)
            new = jnp.maximum(cross, red)
            return new, jnp.any(new != s)

        s, _ = jax.lax.while_loop(cond, body, (cross, jnp.bool_(True)))
        out[0, 0:1, t * _C:(t + 1) * _C] = s
        supp_col[t * _C:(t + 1) * _C, 0:1] = to_col(s)


def _nms_suppress(boxes_off):
    B = boxes_off.shape[0]
    bc = jnp.transpose(boxes_off, (0, 2, 1))
    out = pl.pallas_call(
        _nms_suppress_kernel,
        grid=(B,),
        in_specs=[
            pl.BlockSpec((1, _NPAD, 4), lambda b: (b, 0, 0)),
            pl.BlockSpec((1, 4, _NPAD), lambda b: (b, 0, 0)),
        ],
        out_specs=pl.BlockSpec((1, 1, _NPAD), lambda b: (b, 0, 0)),
        out_shape=jax.ShapeDtypeStruct((B, 1, _NPAD), jnp.float32),
        scratch_shapes=[pltpu.VMEM((_NPAD, 1), jnp.float32)],
    )(boxes_off, bc)
    return out[:, 0, :_N_NMS]


def _decode_clip(rel, anchors):
    w = anchors[..., 2] - anchors[..., 0]
    h = anchors[..., 3] - anchors[..., 1]
    cx = anchors[..., 0] + 0.5 * w
    cy = anchors[..., 1] + 0.5 * h
    dx, dy = rel[..., 0], rel[..., 1]
    dw = jnp.minimum(rel[..., 2], _CLIP)
    dh = jnp.minimum(rel[..., 3], _CLIP)
    pcx = dx * w + cx
    pcy = dy * h + cy
    pw = jnp.exp(dw) * w
    ph = jnp.exp(dh) * h
    x1 = jnp.clip(pcx - 0.5 * pw, 0.0, float(_IMG_W))
    y1 = jnp.clip(pcy - 0.5 * ph, 0.0, float(_IMG_H))
    x2 = jnp.clip(pcx + 0.5 * pw, 0.0, float(_IMG_W))
    y2 = jnp.clip(pcy + 0.5 * ph, 0.0, float(_IMG_H))
    return jnp.stack([x1, y1, x2, y2], axis=-1)


def kernel(image_tensors, feat0, feat1, feat2, feat3, feat4, cls_logits, bbox_regression):
    feats = [feat0, feat1, feat2, feat3, feat4]
    counts = [f.shape[2] * f.shape[3] * _A for f in feats]
    B = cls_logits.shape[0]

    boxes_l, scores_l, labels_l = [], [], []
    off = 0
    for li, cnt in enumerate(counts):
        logits = cls_logits[:, off:off + cnt]
        regress = bbox_regression[:, off:off + cnt]
        off += cnt
        sc = jax.nn.sigmoid(logits).reshape(B, -1)
        masked = jnp.where(sc > _SCORE_THRESH, sc, -1.0)
        top_s, top_i = jax.lax.approx_max_k(masked, _TOPK, recall_target=1.0)
        anchor_idx = top_i // _NUM_CLASSES
        label = top_i % _NUM_CLASSES
        rel = jnp.take_along_axis(regress, anchor_idx[..., None], axis=1)
        anc = jnp.asarray(_ANCHORS_NP[li])[anchor_idx]
        boxes_l.append(_decode_clip(rel, anc))
        scores_l.append(top_s)
        labels_l.append(label)

    boxes = jnp.concatenate(boxes_l, axis=1)
    scores = jnp.concatenate(scores_l, axis=1)
    labels = jnp.concatenate(labels_l, axis=1)

    order = jnp.argsort(-scores, axis=1)
    b = jnp.take_along_axis(boxes, order[..., None], axis=1)
    s = jnp.take_along_axis(scores, order, axis=1)
    l = jnp.take_along_axis(labels, order, axis=1)

    offs = (l.astype(b.dtype) * (float(max(_IMG_H, _IMG_W)) + 1.0))[..., None]
    boff = b + offs
    boff = jnp.pad(boff, ((0, 0), (0, _NPAD - _N_NMS), (0, 0)))
    supp = _nms_suppress(boff)

    kept = jnp.where(supp > 0.5, -1.0, s)
    top_s, top_i = jax.lax.top_k(kept, _DETS)
    out_boxes = jnp.take_along_axis(b, top_i[..., None], axis=1)
    out_labels = jnp.take_along_axis(l, top_i, axis=1)
    det = jnp.concatenate([out_boxes, top_s[..., None]], axis=-1)
    return det, out_labels.astype(jnp.int32)


# speedup vs baseline: 18.2169x; 3.1034x over previous
import functools
import math

import jax
import jax.numpy as jnp
import numpy as np
from jax.experimental import pallas as pl
from jax.experimental.pallas import tpu as pltpu

_IMG_H = 800
_IMG_W = 800
_NUM_CLASSES = 91
_A = 9
_GRIDS = [(100, 100), (50, 50), (25, 25), (13, 13), (7, 7)]
_SIZES = [(32, 40, 50), (64, 80, 101), (128, 161, 203), (256, 322, 406), (512, 645, 812)]
_RATIOS = (0.5, 1.0, 2.0)
_SCORE_THRESH = 0.05
_TOPK = 1000
_NMS_THRESH = 0.5
_DETS = 300
_CLIP = math.log(1000.0 / 16.0)

_N_NMS = 5000
_C = 512
_NPAD = 5120


def _cell_anchors(sizes):
    scales = np.asarray(sizes, dtype=np.float32)
    ratios = np.asarray(_RATIOS, dtype=np.float32)
    h_ratios = np.sqrt(ratios)
    w_ratios = 1.0 / h_ratios
    ws = (w_ratios[:, None] * scales[None, :]).reshape(-1)
    hs = (h_ratios[:, None] * scales[None, :]).reshape(-1)
    base = np.stack([-ws, -hs, ws, hs], axis=1) / 2.0
    return np.round(base).astype(np.float32)


def _make_anchors():
    per_level = []
    for (gh, gw), sz in zip(_GRIDS, _SIZES):
        stride_h = _IMG_H // gh
        stride_w = _IMG_W // gw
        base = _cell_anchors(sz)
        sx = np.arange(gw, dtype=np.float32) * stride_w
        sy = np.arange(gh, dtype=np.float32) * stride_h
        SY, SX = np.meshgrid(sy, sx, indexing="ij")
        shifts = np.stack([SX.reshape(-1), SY.reshape(-1), SX.reshape(-1), SY.reshape(-1)], axis=1)
        a = (shifts[:, None, :] + base[None, :, :]).reshape(-1, 4)
        per_level.append(np.asarray(a, dtype=np.float32))
    return per_level


_ANCHORS_NP = _make_anchors()


def _nms_suppress_kernel(br, bc, out, supp_col):
    nchunks = _NPAD // _C
    ii = jax.lax.broadcasted_iota(jnp.int32, (_C, _C), 0)
    jj = jax.lax.broadcasted_iota(jnp.int32, (_C, _C), 1)
    upper = (jj > ii).astype(jnp.float32)
    ident = (jj == ii).astype(jnp.float32)

    for t in range(nchunks):
        x1t = bc[0, 0:1, t * _C:(t + 1) * _C]
        y1t = bc[0, 1:2, t * _C:(t + 1) * _C]
        x2t = bc[0, 2:3, t * _C:(t + 1) * _C]
        y2t = bc[0, 3:4, t * _C:(t + 1) * _C]
        area_t = (x2t - x1t) * (y2t - y1t)

        cross = jnp.zeros((1, _C), jnp.float32)
        adj = None
        for p in range(t + 1):
            x1p = br[0, p * _C:(p + 1) * _C, 0:1]
            y1p = br[0, p * _C:(p + 1) * _C, 1:2]
            x2p = br[0, p * _C:(p + 1) * _C, 2:3]
            y2p = br[0, p * _C:(p + 1) * _C, 3:4]
            area_p = (x2p - x1p) * (y2p - y1p)
            w = jnp.maximum(jnp.minimum(x2p, x2t) - jnp.maximum(x1p, x1t), 0.0)
            h = jnp.maximum(jnp.minimum(y2p, y2t) - jnp.maximum(y1p, y1t), 0.0)
            inter = w * h
            iou = inter / (area_p + area_t - inter + 1e-7)
            a = (iou > _NMS_THRESH).astype(jnp.float32)
            if p < t:
                kept_p = 1.0 - supp_col[p * _C:(p + 1) * _C, 0:1]
                cross = jnp.maximum(cross, jnp.max(a * kept_p, axis=0, keepdims=True))
            else:
                adj = a * upper

        def to_col(row):
            return jax.lax.dot_general(ident, row,
                                       (((1,), (1,)), ((), ())),
                                       preferred_element_type=jnp.float32)

        def cond(carry):
            return carry[1]

        def body(carry):
            s, _ = carry
            s_col = to_col(s)
            red = jnp.max(adj * (1.0 - s_col), axis=0, keepdims=True)
            new = jnp.maximum(cross, red)
            return new, jnp.any(new != s)

        s, _ = jax.lax.while_loop(cond, body, (cross, jnp.bool_(True)))
        out[0, 0:1, t * _C:(t + 1) * _C] = s
        supp_col[t * _C:(t + 1) * _C, 0:1] = to_col(s)


def _nms_suppress(boxes_off):
    B = boxes_off.shape[0]
    bc = jnp.transpose(boxes_off, (0, 2, 1))
    out = pl.pallas_call(
        _nms_suppress_kernel,
        grid=(B,),
        in_specs=[
            pl.BlockSpec((1, _NPAD, 4), lambda b: (b, 0, 0)),
            pl.BlockSpec((1, 4, _NPAD), lambda b: (b, 0, 0)),
        ],
        out_specs=pl.BlockSpec((1, 1, _NPAD), lambda b: (b, 0, 0)),
        out_shape=jax.ShapeDtypeStruct((B, 1, _NPAD), jnp.float32),
        scratch_shapes=[pltpu.VMEM((_NPAD, 1), jnp.float32)],
    )(boxes_off, bc)
    return out[:, 0, :_N_NMS]


def _decode_clip(rel, anchors):
    w = anchors[..., 2] - anchors[..., 0]
    h = anchors[..., 3] - anchors[..., 1]
    cx = anchors[..., 0] + 0.5 * w
    cy = anchors[..., 1] + 0.5 * h
    dx, dy = rel[..., 0], rel[..., 1]
    dw = jnp.minimum(rel[..., 2], _CLIP)
    dh = jnp.minimum(rel[..., 3], _CLIP)
    pcx = dx * w + cx
    pcy = dy * h + cy
    pw = jnp.exp(dw) * w
    ph = jnp.exp(dh) * h
    x1 = jnp.clip(pcx - 0.5 * pw, 0.0, float(_IMG_W))
    y1 = jnp.clip(pcy - 0.5 * ph, 0.0, float(_IMG_H))
    x2 = jnp.clip(pcx + 0.5 * pw, 0.0, float(_IMG_W))
    y2 = jnp.clip(pcy + 0.5 * ph, 0.0, float(_IMG_H))
    return jnp.stack([x1, y1, x2, y2], axis=-1)


_LOGIT_T = float(math.log(_SCORE_THRESH / (1.0 - _SCORE_THRESH)))
_ROWKEY_RB = 1024
_PRESEL = 1024


def _rowkey_kernel(x_ref, o_ref):
    x = x_ref[0]
    key = jnp.where(x > (_LOGIT_T - 1e-3), x, -1e30)
    o_ref[0] = jnp.max(key, axis=1, keepdims=True)


def _anchor_keys(cls_logits):
    B, hwa, nc = cls_logits.shape
    g = -(-hwa // _ROWKEY_RB)
    out = pl.pallas_call(
        _rowkey_kernel,
        grid=(B, g),
        in_specs=[pl.BlockSpec((1, _ROWKEY_RB, nc), lambda b, i: (b, i, 0))],
        out_specs=pl.BlockSpec((1, _ROWKEY_RB, 1), lambda b, i: (b, i, 0)),
        out_shape=jax.ShapeDtypeStruct((B, g * _ROWKEY_RB, 1), jnp.float32),
    )(cls_logits)
    return out[:, :hwa, 0]


def _level_select_slow(logits, B):
    sc = jax.nn.sigmoid(logits).reshape(B, -1)
    masked = jnp.where(sc > _SCORE_THRESH, sc, -1.0)
    top_s, top_i = jax.lax.top_k(masked, _TOPK)
    return top_s, top_i // _NUM_CLASSES, top_i % _NUM_CLASSES


def _level_select_fast(logits, km, B):
    _, aidx = jax.lax.top_k(km, _PRESEL)
    aidx = jnp.sort(aidx, axis=1)
    g = jnp.take_along_axis(logits, aidx[..., None], axis=1)
    sc = jax.nn.sigmoid(g).reshape(B, -1)
    masked = jnp.where(sc > _SCORE_THRESH, sc, -1.0)
    top_s, ti = jax.lax.top_k(masked, _TOPK)
    anchor_idx = jnp.take_along_axis(aidx, ti // _NUM_CLASSES, axis=1)
    label = ti % _NUM_CLASSES
    return top_s, anchor_idx, label


def kernel(image_tensors, feat0, feat1, feat2, feat3, feat4, cls_logits, bbox_regression):
    feats = [feat0, feat1, feat2, feat3, feat4]
    counts = [f.shape[2] * f.shape[3] * _A for f in feats]
    B = cls_logits.shape[0]

    rowkey = _anchor_keys(cls_logits)

    boxes_l, scores_l, labels_l = [], [], []
    off = 0
    for li, cnt in enumerate(counts):
        logits = cls_logits[:, off:off + cnt]
        regress = bbox_regression[:, off:off + cnt]
        km = rowkey[:, off:off + cnt]
        off += cnt
        if cnt > _PRESEL:
            top_s, anchor_idx, label = _level_select_fast(logits, km, B)
            need_slow = jnp.any(top_s[:, _TOPK - 1] < 0.0)
            top_s, anchor_idx, label = jax.lax.cond(
                need_slow,
                lambda ops: _level_select_slow(logits, B),
                lambda ops: ops,
                (top_s, anchor_idx, label))
        else:
            top_s, anchor_idx, label = _level_select_slow(logits, B)
        rel = jnp.take_along_axis(regress, anchor_idx[..., None], axis=1)
        anc = jnp.asarray(_ANCHORS_NP[li])[anchor_idx]
        boxes_l.append(_decode_clip(rel, anc))
        scores_l.append(top_s)
        labels_l.append(label)

    boxes = jnp.concatenate(boxes_l, axis=1)
    scores = jnp.concatenate(scores_l, axis=1)
    labels = jnp.concatenate(labels_l, axis=1)

    order = jnp.argsort(-scores, axis=1)
    b = jnp.take_along_axis(boxes, order[..., None], axis=1)
    s = jnp.take_along_axis(scores, order, axis=1)
    l = jnp.take_along_axis(labels, order, axis=1)

    offs = (l.astype(b.dtype) * (float(max(_IMG_H, _IMG_W)) + 1.0))[..., None]
    boff = b + offs
    boff = jnp.pad(boff, ((0, 0), (0, _NPAD - _N_NMS), (0, 0)))
    supp = _nms_suppress(boff)

    kept = jnp.where(supp > 0.5, -1.0, s)
    top_s, top_i = jax.lax.top_k(kept, _DETS)
    out_boxes = jnp.take_along_axis(b, top_i[..., None], axis=1)
    out_labels = jnp.take_along_axis(l, top_i, axis=1)
    det = jnp.concatenate([out_boxes, top_s[..., None]], axis=-1)
    return det, out_labels.astype(jnp.int32)


# speedup vs baseline: 29.6366x; 1.6269x over previous
import functools
import math

import jax
import jax.numpy as jnp
import numpy as np
from jax.experimental import pallas as pl
from jax.experimental.pallas import tpu as pltpu

_IMG_H = 800
_IMG_W = 800
_NUM_CLASSES = 91
_A = 9
_GRIDS = [(100, 100), (50, 50), (25, 25), (13, 13), (7, 7)]
_SIZES = [(32, 40, 50), (64, 80, 101), (128, 161, 203), (256, 322, 406), (512, 645, 812)]
_RATIOS = (0.5, 1.0, 2.0)
_SCORE_THRESH = 0.05
_TOPK = 1000
_NMS_THRESH = 0.5
_DETS = 300
_CLIP = math.log(1000.0 / 16.0)

_N_NMS = 5000
_C = 512
_NPAD = 5120


def _cell_anchors(sizes):
    scales = np.asarray(sizes, dtype=np.float32)
    ratios = np.asarray(_RATIOS, dtype=np.float32)
    h_ratios = np.sqrt(ratios)
    w_ratios = 1.0 / h_ratios
    ws = (w_ratios[:, None] * scales[None, :]).reshape(-1)
    hs = (h_ratios[:, None] * scales[None, :]).reshape(-1)
    base = np.stack([-ws, -hs, ws, hs], axis=1) / 2.0
    return np.round(base).astype(np.float32)


def _make_anchors():
    per_level = []
    for (gh, gw), sz in zip(_GRIDS, _SIZES):
        stride_h = _IMG_H // gh
        stride_w = _IMG_W // gw
        base = _cell_anchors(sz)
        sx = np.arange(gw, dtype=np.float32) * stride_w
        sy = np.arange(gh, dtype=np.float32) * stride_h
        SY, SX = np.meshgrid(sy, sx, indexing="ij")
        shifts = np.stack([SX.reshape(-1), SY.reshape(-1), SX.reshape(-1), SY.reshape(-1)], axis=1)
        a = (shifts[:, None, :] + base[None, :, :]).reshape(-1, 4)
        per_level.append(np.asarray(a, dtype=np.float32))
    return per_level


_ANCHORS_NP = _make_anchors()


def _nms_suppress_kernel(br, bc, out, supp_col):
    nchunks = _NPAD // _C
    ii = jax.lax.broadcasted_iota(jnp.int32, (_C, _C), 0)
    jj = jax.lax.broadcasted_iota(jnp.int32, (_C, _C), 1)
    upper = (jj > ii).astype(jnp.float32)
    ident = (jj == ii).astype(jnp.float32)

    for t in range(nchunks):
        x1t = bc[0, 0:1, t * _C:(t + 1) * _C]
        y1t = bc[0, 1:2, t * _C:(t + 1) * _C]
        x2t = bc[0, 2:3, t * _C:(t + 1) * _C]
        y2t = bc[0, 3:4, t * _C:(t + 1) * _C]
        area_t = (x2t - x1t) * (y2t - y1t)

        cross = jnp.zeros((1, _C), jnp.float32)
        adj = None
        for p in range(t + 1):
            x1p = br[0, p * _C:(p + 1) * _C, 0:1]
            y1p = br[0, p * _C:(p + 1) * _C, 1:2]
            x2p = br[0, p * _C:(p + 1) * _C, 2:3]
            y2p = br[0, p * _C:(p + 1) * _C, 3:4]
            area_p = (x2p - x1p) * (y2p - y1p)
            w = jnp.maximum(jnp.minimum(x2p, x2t) - jnp.maximum(x1p, x1t), 0.0)
            h = jnp.maximum(jnp.minimum(y2p, y2t) - jnp.maximum(y1p, y1t), 0.0)
            inter = w * h
            iou = inter / (area_p + area_t - inter + 1e-7)
            a = (iou > _NMS_THRESH).astype(jnp.float32)
            if p < t:
                kept_p = 1.0 - supp_col[p * _C:(p + 1) * _C, 0:1]
                cross = jnp.maximum(cross, jnp.max(a * kept_p, axis=0, keepdims=True))
            else:
                adj = a * upper

        def to_col(row):
            return jax.lax.dot_general(ident, row,
                                       (((1,), (1,)), ((), ())),
                                       preferred_element_type=jnp.float32)

        def cond(carry):
            return carry[1]

        def body(carry):
            s, _ = carry
            s_col = to_col(s)
            red = jnp.max(adj * (1.0 - s_col), axis=0, keepdims=True)
            new = jnp.maximum(cross, red)
            return new, jnp.any(new != s)

        s, _ = jax.lax.while_loop(cond, body, (cross, jnp.bool_(True)))
        out[0, 0:1, t * _C:(t + 1) * _C] = s
        supp_col[t * _C:(t + 1) * _C, 0:1] = to_col(s)


def _nms_suppress(boxes_off):
    B = boxes_off.shape[0]
    bc = jnp.transpose(boxes_off, (0, 2, 1))
    out = pl.pallas_call(
        _nms_suppress_kernel,
        grid=(B,),
        in_specs=[
            pl.BlockSpec((1, _NPAD, 4), lambda b: (b, 0, 0)),
            pl.BlockSpec((1, 4, _NPAD), lambda b: (b, 0, 0)),
        ],
        out_specs=pl.BlockSpec((1, 1, _NPAD), lambda b: (b, 0, 0)),
        out_shape=jax.ShapeDtypeStruct((B, 1, _NPAD), jnp.float32),
        scratch_shapes=[pltpu.VMEM((_NPAD, 1), jnp.float32)],
    )(boxes_off, bc)
    return out[:, 0, :_N_NMS]


def _decode_clip(rel, anchors):
    w = anchors[..., 2] - anchors[..., 0]
    h = anchors[..., 3] - anchors[..., 1]
    cx = anchors[..., 0] + 0.5 * w
    cy = anchors[..., 1] + 0.5 * h
    dx, dy = rel[..., 0], rel[..., 1]
    dw = jnp.minimum(rel[..., 2], _CLIP)
    dh = jnp.minimum(rel[..., 3], _CLIP)
    pcx = dx * w + cx
    pcy = dy * h + cy
    pw = jnp.exp(dw) * w
    ph = jnp.exp(dh) * h
    x1 = jnp.clip(pcx - 0.5 * pw, 0.0, float(_IMG_W))
    y1 = jnp.clip(pcy - 0.5 * ph, 0.0, float(_IMG_H))
    x2 = jnp.clip(pcx + 0.5 * pw, 0.0, float(_IMG_W))
    y2 = jnp.clip(pcy + 0.5 * ph, 0.0, float(_IMG_H))
    return jnp.stack([x1, y1, x2, y2], axis=-1)


_LOGIT_T = float(math.log(_SCORE_THRESH / (1.0 - _SCORE_THRESH)))
_ROWKEY_RB = 1024
_PRESEL = 1024


def _rowkey_kernel(x_ref, o_ref):
    x = x_ref[0]
    key = jnp.where(x > (_LOGIT_T - 1e-3), x, -1e30)
    o_ref[0] = jnp.max(key, axis=1, keepdims=True)


def _anchor_keys(cls_logits):
    B, hwa, nc = cls_logits.shape
    g = -(-hwa // _ROWKEY_RB)
    out = pl.pallas_call(
        _rowkey_kernel,
        grid=(B, g),
        in_specs=[pl.BlockSpec((1, _ROWKEY_RB, nc), lambda b, i: (b, i, 0))],
        out_specs=pl.BlockSpec((1, _ROWKEY_RB, 1), lambda b, i: (b, i, 0)),
        out_shape=jax.ShapeDtypeStruct((B, g * _ROWKEY_RB, 1), jnp.float32),
    )(cls_logits)
    return out[:, :hwa, 0]


def _level_select_slow(logits, B):
    sc = jax.nn.sigmoid(logits).reshape(B, -1)
    masked = jnp.where(sc > _SCORE_THRESH, sc, -1.0)
    top_s, top_i = jax.lax.top_k(masked, _TOPK)
    return top_s, top_i // _NUM_CLASSES, top_i % _NUM_CLASSES


def kernel(image_tensors, feat0, feat1, feat2, feat3, feat4, cls_logits, bbox_regression):
    feats = [feat0, feat1, feat2, feat3, feat4]
    counts = [f.shape[2] * f.shape[3] * _A for f in feats]
    B = cls_logits.shape[0]
    offs = np.concatenate([[0], np.cumsum(counts)]).astype(np.int32)

    rowkey = _anchor_keys(cls_logits)

    fast = [li for li, cnt in enumerate(counts) if cnt > _PRESEL]
    slow = [li for li, cnt in enumerate(counts) if cnt <= _PRESEL]
    km_pad = max(counts[li] for li in fast)

    km_stack = jnp.stack(
        [jnp.pad(rowkey[:, offs[li]:offs[li + 1]],
                 ((0, 0), (0, km_pad - counts[li])), constant_values=-2e30)
         for li in fast], axis=0)
    _, aidx = jax.lax.top_k(km_stack.reshape(len(fast) * B, km_pad), _PRESEL)
    aidx = jnp.sort(aidx, axis=1).reshape(len(fast), B, _PRESEL)

    fin_w = _PRESEL * _NUM_CLASSES
    fin_rows = []
    for i, li in enumerate(fast):
        g = jnp.take_along_axis(cls_logits[:, offs[li]:offs[li + 1]],
                                aidx[i][..., None], axis=1)
        sc = jax.nn.sigmoid(g).reshape(B, -1)
        fin_rows.append(jnp.where(sc > _SCORE_THRESH, sc, -1.0))
    for li in slow:
        sc = jax.nn.sigmoid(cls_logits[:, offs[li]:offs[li + 1]]).reshape(B, -1)
        m = jnp.where(sc > _SCORE_THRESH, sc, -1.0)
        fin_rows.append(jnp.pad(m, ((0, 0), (0, fin_w - m.shape[1])),
                                constant_values=-2.0))

    fin = jnp.stack(fin_rows, axis=0).reshape(len(counts) * B, fin_w)
    ts_all, ti_all = jax.lax.top_k(fin, _TOPK)
    ts_all = ts_all.reshape(len(counts), B, _TOPK)
    ti_all = ti_all.reshape(len(counts), B, _TOPK)

    scores_l = [None] * len(counts)
    labels_l = [None] * len(counts)
    aidx_g = [None] * len(counts)
    for i, li in enumerate(fast):
        top_s, ti = ts_all[i], ti_all[i]
        anchor_idx = jnp.take_along_axis(aidx[i], ti // _NUM_CLASSES, axis=1)
        label = ti % _NUM_CLASSES
        need_slow = jnp.any(top_s[:, _TOPK - 1] < 0.0)
        lg = cls_logits[:, offs[li]:offs[li + 1]]
        top_s, anchor_idx, label = jax.lax.cond(
            need_slow,
            lambda ops, lg=lg: _level_select_slow(lg, B),
            lambda ops: ops,
            (top_s, anchor_idx, label))
        scores_l[li] = top_s
        labels_l[li] = label
        aidx_g[li] = anchor_idx + offs[li]
    for j, li in enumerate(slow):
        top_s, ti = ts_all[len(fast) + j], ti_all[len(fast) + j]
        scores_l[li] = top_s
        labels_l[li] = ti % _NUM_CLASSES
        aidx_g[li] = ti // _NUM_CLASSES + offs[li]

    anchor_all = jnp.concatenate(aidx_g, axis=1)
    rel = jnp.take_along_axis(bbox_regression, anchor_all[..., None], axis=1)
    anc = jnp.asarray(np.concatenate(_ANCHORS_NP, axis=0))[anchor_all]
    boxes = _decode_clip(rel, anc)
    scores = jnp.concatenate(scores_l, axis=1)
    labels = jnp.concatenate(labels_l, axis=1)

    order = jnp.argsort(-scores, axis=1)
    b = jnp.take_along_axis(boxes, order[..., None], axis=1)
    s = jnp.take_along_axis(scores, order, axis=1)
    l = jnp.take_along_axis(labels, order, axis=1)

    offs = (l.astype(b.dtype) * (float(max(_IMG_H, _IMG_W)) + 1.0))[..., None]
    boff = b + offs
    boff = jnp.pad(boff, ((0, 0), (0, _NPAD - _N_NMS), (0, 0)))
    supp = _nms_suppress(boff)

    kept = jnp.where(supp > 0.5, -1.0, s)
    top_s, top_i = jax.lax.top_k(kept, _DETS)
    out_boxes = jnp.take_along_axis(b, top_i[..., None], axis=1)
    out_labels = jnp.take_along_axis(l, top_i, axis=1)
    det = jnp.concatenate([out_boxes, top_s[..., None]], axis=-1)
    return det, out_labels.astype(jnp.int32)
